# Initial kernel scaffold; baseline (speedup 1.0000x reference)
import jax
import jax.numpy as jnp
from jax.experimental import pallas as pl


def kernel(token_ids, embed):
    raise NotImplementedError("write your pallas kernel here")


# speedup vs baseline: 1.3066x; 1.3066x over previous
import functools

import jax
import jax.numpy as jnp
from jax import lax
from jax.experimental import pallas as pl
from jax.experimental.pallas import tpu as pltpu
from jax.experimental.pallas import tpu_sc as plsc

NUM_EMB = 1_000_000
DIM = 32
CHUNK = 128


def _make_gather(B: int):
    info = plsc.get_sparse_core_info()
    NC, NS = info.num_cores, info.num_subcores
    NW = NC * NS
    assert B % (NW * CHUNK) == 0
    n_chunks = B // (NW * CHUNK)

    mesh = plsc.VectorSubcoreMesh(core_axis_name="c", subcore_axis_name="s")

    @functools.partial(
        pl.kernel,
        out_type=jax.ShapeDtypeStruct((B, DIM), jnp.float32),
        mesh=mesh,
        scratch_types=[
            pltpu.VMEM((n_chunks, CHUNK), jnp.int32),
            pltpu.VMEM((CHUNK, DIM), jnp.float32),
            pltpu.SemaphoreType.DMA,
        ],
        compiler_params=pltpu.CompilerParams(use_tc_tiling_on_sc=False),
    )
    def emb(idx_hbm, table_hbm, out_hbm, idx_v, rows_v, sem):
        wid = lax.axis_index("s") * NC + lax.axis_index("c")
        pltpu.sync_copy(idx_hbm.at[pl.ds(wid * n_chunks, n_chunks)], idx_v)

        def body(c, carry):
            pltpu.async_copy(table_hbm.at[idx_v.at[c]], rows_v, sem).wait()
            row0 = (wid * n_chunks + c) * CHUNK
            pltpu.sync_copy(rows_v, out_hbm.at[pl.ds(row0, CHUNK)])
            return carry

        lax.fori_loop(0, n_chunks, body, 0)

    return emb


def kernel(token_ids, embed):
    B = token_ids.shape[0] * token_ids.shape[1]
    idx = token_ids.reshape(B // CHUNK, CHUNK).astype(jnp.int32)
    out = _make_gather(B)(idx, embed)
    return out.reshape(token_ids.shape[0], token_ids.shape[1], DIM)


# speedup vs baseline: 1.5020x; 1.1496x over previous
import functools

import jax
import jax.numpy as jnp
from jax import lax
from jax.experimental import pallas as pl
from jax.experimental.pallas import tpu as pltpu
from jax.experimental.pallas import tpu_sc as plsc

DIM = 32
CHUNK = 128
K = 10


def _make_gather(B: int):
    info = plsc.get_sparse_core_info()
    NC, NS = info.num_cores, info.num_subcores
    NW = NC * NS
    assert B % (NW * CHUNK) == 0
    n_chunks = B // (NW * CHUNK)
    assert n_chunks % (2 * K) == 0
    n_groups = n_chunks // K

    mesh = plsc.VectorSubcoreMesh(core_axis_name="c", subcore_axis_name="s")

    @functools.partial(
        pl.kernel,
        out_type=jax.ShapeDtypeStruct((B // CHUNK, CHUNK, DIM), jnp.float32),
        mesh=mesh,
        scratch_types=[
            pltpu.VMEM((n_chunks, CHUNK), jnp.int32),
            pltpu.VMEM((K, CHUNK, DIM), jnp.float32),
            pltpu.VMEM((K, CHUNK, DIM), jnp.float32),
            pltpu.SemaphoreType.DMA,
            pltpu.SemaphoreType.DMA,
        ],
        compiler_params=pltpu.CompilerParams(use_tc_tiling_on_sc=False),
    )
    def emb(idx_hbm, table_hbm, out_hbm, idx_v, buf0, buf1, sem0, sem1):
        wid = lax.axis_index("s") * NC + lax.axis_index("c")
        pltpu.sync_copy(idx_hbm.at[pl.ds(wid * n_chunks, n_chunks)], idx_v)

        def fire(grp, buf, sem):
            for j in range(K):
                pltpu.async_copy(
                    table_hbm.at[idx_v.at[grp * K + j]], buf.at[j], sem
                )

        def drain(buf, sem):
            for j in range(K):
                pltpu.make_async_copy(
                    table_hbm.at[idx_v.at[j]], buf.at[j], sem
                ).wait()

        def writeback(grp, buf):
            c0 = wid * n_chunks + grp * K
            pltpu.sync_copy(buf, out_hbm.at[pl.ds(c0, K)])

        fire(0, buf0, sem0)

        def body(g, carry):
            fire(2 * g + 1, buf1, sem1)
            drain(buf0, sem0)
            writeback(2 * g, buf0)

            @pl.when(2 * g + 2 < n_groups)
            def _():
                fire(2 * g + 2, buf0, sem0)

            drain(buf1, sem1)
            writeback(2 * g + 1, buf1)
            return carry

        lax.fori_loop(0, n_groups // 2, body, 0)

    return emb


def kernel(token_ids, embed):
    B = token_ids.shape[0] * token_ids.shape[1]
    idx = token_ids.reshape(B // CHUNK, CHUNK).astype(jnp.int32)
    out = _make_gather(B)(idx, embed)
    return out.reshape(token_ids.shape[0], token_ids.shape[1], DIM)
